# baseline (speedup 1.0000x reference)
import functools

import jax
import jax.numpy as jnp
from jax import lax
from jax.experimental import pallas as pl
from jax.experimental.pallas import tpu as pltpu
from jax.experimental.pallas import tpu_sc as plsc

COMMITMENT_COST = 0.25
EPSILON = 1e-10


def _vq_block_kernel(x_ref, w_ref, dist_ref, enc_ref, idx_ref,
                     loss_ref, perp_ref, wneg2_ref, w2_ref, counts_ref,
                     cnt_ref, ssq_ref):
    step = pl.program_id(0)
    nsteps = pl.num_programs(0)
    xb = x_ref[...]
    bm = xb.shape[0]
    n = w_ref.shape[1]

    @pl.when(step == 0)
    def _prep():
        wm0 = w_ref[...]
        wneg2_ref[...] = wm0 * (-2.0)
        w2_ref[...] = jnp.sum(wm0 * wm0, axis=0, keepdims=True)

    x2 = jnp.sum(xb * xb, axis=1, keepdims=True)
    mm2 = jnp.dot(xb, wneg2_ref[...],
                  preferred_element_type=jnp.float32)
    d = (x2 + mm2) + w2_ref[...]
    dist_ref[...] = d

    mn = jnp.min(d, axis=1, keepdims=True)
    maskb = d == mn
    iota = jax.lax.broadcasted_iota(jnp.int32, (bm, n), 1)
    idx = jnp.min(jnp.where(maskb, iota, n), axis=1, keepdims=True)
    idx_ref[...] = idx

    enc_ref[...] = maskb.astype(jnp.float32)
    ones_row = jnp.full((1, bm), 1.0, jnp.float32)
    cnt = jnp.dot(ones_row, enc_ref[...],
                  preferred_element_type=jnp.float32)
    cnt_ref[...] = cnt

    tie = jnp.sum(cnt) > jnp.float32(bm) + 0.5

    @pl.when(tie)
    def _fix():
        e = (iota == idx).astype(jnp.float32)
        enc_ref[...] = e
        cnt_ref[...] = jnp.dot(ones_row, e,
                               preferred_element_type=jnp.float32)

    ssq = jnp.sum(mn).reshape(1, 1)

    @pl.when(step == 0)
    def _init():
        counts_ref[...] = cnt_ref[...]
        ssq_ref[...] = ssq

    @pl.when(step > 0)
    def _acc():
        counts_ref[...] += cnt_ref[...]
        ssq_ref[...] += ssq

    @pl.when(step == nsteps - 1)
    def _fin():
        total = jnp.float32(bm) * nsteps
        avg = counts_ref[...] / total
        ent = -jnp.sum(avg * jnp.log(avg + EPSILON))
        perp_ref[...] = jnp.exp(ent).reshape(1, 1)
        scale = (1.0 + COMMITMENT_COST) / (total * xb.shape[1])
        loss_ref[...] = ssq_ref[...] * scale


def _sc_gather(table, idx, b, d):
    info = plsc.get_sparse_core_info()
    nw = info.num_cores * info.num_subcores
    b_per_w = b // nw
    chunk = 64
    nchunks = b_per_w // chunk
    mesh = plsc.VectorSubcoreMesh(core_axis_name="c", subcore_axis_name="s")

    @functools.partial(
        pl.kernel, mesh=mesh,
        out_type=jax.ShapeDtypeStruct((b, d), jnp.float32),
        scratch_types=[
            pltpu.VMEM((chunk,), jnp.int32),
            pltpu.VMEM((chunk, d), jnp.float32),
            pltpu.SemaphoreType.DMA,
        ],
    )
    def k(table_hbm, idx_hbm, out_hbm, idx_v, rows_v, sem):
        wid = lax.axis_index("s") * info.num_cores + lax.axis_index("c")
        base = wid * b_per_w

        def body(c, carry):
            off = base + c * chunk
            pltpu.sync_copy(idx_hbm.at[pl.ds(off, chunk)], idx_v)
            pltpu.async_copy(table_hbm.at[idx_v], rows_v, sem).wait()
            pltpu.sync_copy(rows_v, out_hbm.at[pl.ds(off, chunk)])
            return carry

        lax.fori_loop(0, nchunks, body, 0)

    return k(table, idx)


def kernel(x, w):
    k = w.shape[0]
    n = w.shape[1]
    xf = x.reshape(-1, k)
    m = xf.shape[0]
    bm = 256 if m % 256 == 0 else m
    grid = m // bm

    out_types = (
        jax.ShapeDtypeStruct((m, n), jnp.float32),
        jax.ShapeDtypeStruct((m, n), jnp.float32),
        jax.ShapeDtypeStruct((m, 1), jnp.int32),
        jax.ShapeDtypeStruct((1, 1), jnp.float32),
        jax.ShapeDtypeStruct((1, 1), jnp.float32),
    )
    dist, enc, idx, loss, perp = pl.pallas_call(
        _vq_block_kernel,
        grid=(grid,),
        in_specs=[
            pl.BlockSpec((bm, k), lambda i: (i, 0)),
            pl.BlockSpec((k, n), lambda i: (0, 0)),
        ],
        out_specs=(
            pl.BlockSpec((bm, n), lambda i: (i, 0)),
            pl.BlockSpec((bm, n), lambda i: (i, 0)),
            pl.BlockSpec((bm, 1), lambda i: (i, 0)),
            pl.BlockSpec((1, 1), lambda i: (0, 0)),
            pl.BlockSpec((1, 1), lambda i: (0, 0)),
        ),
        out_shape=out_types,
        scratch_shapes=[
            pltpu.VMEM((k, n), jnp.float32),
            pltpu.VMEM((1, n), jnp.float32),
            pltpu.VMEM((1, n), jnp.float32),
            pltpu.VMEM((1, n), jnp.float32),
            pltpu.VMEM((1, 1), jnp.float32),
        ],
    )(xf, w)

    q = _sc_gather(w.T, idx.reshape(m), m, k)
    quantized_st = q.reshape(x.shape)
    encoding_indices = idx.reshape(x.shape[:-1])
    return (quantized_st, loss[0, 0], perp[0, 0], enc, encoding_indices, dist)
